# Initial kernel scaffold; baseline (speedup 1.0000x reference)
import functools

import jax
import jax.numpy as jnp
from jax import lax
from jax.experimental import pallas as pl
from jax.experimental.pallas import tpu as pltpu
from jax.experimental.pallas import tpu_sc as plsc

B = 16384
F = 26
V = 100000
D = 64
NC, NS, L = 2, 16, 16
NW = NC * NS
SPW = B // NW
C = 32
NCH = SPW // C
RPC = C * F
NSTREAM = RPC // 64
XROWS_PW = SPW * F // 64

_mesh = plsc.VectorSubcoreMesh(core_axis_name="c", subcore_axis_name="s")


@functools.partial(
    pl.kernel,
    mesh=_mesh,
    out_type=jax.ShapeDtypeStruct((B, D), jnp.float32),
    scratch_types=[
        pltpu.VMEM((NSTREAM, 64), jnp.int32),
        pltpu.VMEM((NSTREAM, 64), jnp.int32),
        pltpu.VMEM((RPC, D), jnp.float32),
        pltpu.VMEM((C, D), jnp.float32),
        pltpu.SemaphoreType.DMA,
    ],
)
def _emb(x_hbm, tab_hbm, out_hbm, idx_v, offs_v, rows_v, out_v, sem):
    wid = lax.axis_index("s") * NC + lax.axis_index("c")

    for k in range(RPC // L):
        p = lax.iota(jnp.int32, L) + (k * L)
        offs_v[k // 4, pl.ds((k % 4) * L, L)] = lax.rem(p, F) * V

    @pl.loop(0, NCH)
    def _chunk(c):
        xrow = wid * XROWS_PW + c * NSTREAM
        pltpu.sync_copy(x_hbm.at[pl.ds(xrow, NSTREAM), :], idx_v)
        for k in range(RPC // L):
            r, col = k // 4, (k % 4) * L
            idx_v[r, pl.ds(col, L)] = idx_v[r, pl.ds(col, L)] + offs_v[r, pl.ds(col, L)]
        copies = [
            pltpu.async_copy(
                tab_hbm.at[idx_v.at[j]], rows_v.at[pl.ds(j * 64, 64), :], sem
            )
            for j in range(NSTREAM)
        ]
        for cp in copies:
            cp.wait()

        @pl.loop(0, C)
        def _acc(s):
            base = s * F
            for v in range(D // L):
                sl = pl.ds(v * L, L)
                a = rows_v[base, sl]
                for i in range(1, F):
                    a = a + rows_v[base + i, sl]
                out_v[s, sl] = a

        pltpu.sync_copy(out_v, out_hbm.at[pl.ds(wid * SPW + c * C, C), :])


def kernel(x, tables):
    x2 = x.astype(jnp.int32).reshape(B * F // 64, 64)
    tab = tables.reshape(F * V, D)
    return _emb(x2, tab)


# speedup vs baseline: 1.0117x; 1.0117x over previous
import functools

import jax
import jax.numpy as jnp
from jax import lax
from jax.experimental import pallas as pl
from jax.experimental.pallas import tpu as pltpu
from jax.experimental.pallas import tpu_sc as plsc

B = 16384
F = 26
V = 100000
D = 64
NC, NS, L = 2, 16, 16
NW = NC * NS
SPW = B // NW
C = 32
NCH = SPW // C
RPC = C * F
NSTREAM = RPC // 64
XROWS_PW = SPW * F // 64

_mesh = plsc.VectorSubcoreMesh(core_axis_name="c", subcore_axis_name="s")


@functools.partial(
    pl.kernel,
    mesh=_mesh,
    out_type=jax.ShapeDtypeStruct((B, D), jnp.float32),
    scratch_types=[
        pltpu.VMEM((XROWS_PW, 64), jnp.int32),
        pltpu.VMEM((NSTREAM, 64), jnp.int32),
        pltpu.VMEM((RPC, D), jnp.float32),
        pltpu.VMEM((C, D), jnp.float32),
        pltpu.SemaphoreType.DMA,
    ],
    compiler_params=pltpu.CompilerParams(use_tc_tiling_on_sc=False),
)
def _emb(x_hbm, tab_hbm, out_hbm, idx_v, offs_v, rows_v, out_v, sem):
    wid = lax.axis_index("s") * NC + lax.axis_index("c")

    for k in range(RPC // L):
        p = lax.iota(jnp.int32, L) + (k * L)
        offs_v[k // 4, pl.ds((k % 4) * L, L)] = lax.rem(p, F) * V

    pltpu.sync_copy(x_hbm.at[pl.ds(wid * XROWS_PW, XROWS_PW), :], idx_v)

    @pl.loop(0, NCH)
    def _offadd(c):
        for k in range(RPC // L):
            r, col = k // 4, (k % 4) * L
            row = c * NSTREAM + r
            idx_v[row, pl.ds(col, L)] = (
                idx_v[row, pl.ds(col, L)] + offs_v[r, pl.ds(col, L)]
            )

    @pl.loop(0, NCH)
    def _chunk(c):
        copies = [
            pltpu.async_copy(
                tab_hbm.at[idx_v.at[c * NSTREAM + j]],
                rows_v.at[pl.ds(j * 64, 64), :],
                sem,
            )
            for j in range(NSTREAM)
        ]
        for cp in copies:
            cp.wait()

        @pl.loop(0, C)
        def _acc(s):
            base = s * F
            for v in range(D // L):
                sl = pl.ds(v * L, L)
                a = rows_v[base, sl]
                for i in range(1, F):
                    a = a + rows_v[base + i, sl]
                out_v[s, sl] = a

        pltpu.sync_copy(out_v, out_hbm.at[pl.ds(wid * SPW + c * C, C), :])


def kernel(x, tables):
    x2 = x.astype(jnp.int32).reshape(B * F // 64, 64)
    tab = tables.reshape(F * V, D)
    return _emb(x2, tab)


# speedup vs baseline: 1.0400x; 1.0280x over previous
import functools

import jax
import jax.numpy as jnp
from jax import lax
from jax.experimental import pallas as pl
from jax.experimental.pallas import tpu as pltpu
from jax.experimental.pallas import tpu_sc as plsc

B = 16384
F = 26
V = 100000
D = 64
NC, NS, L = 2, 16, 16
NW = NC * NS
SPW = B // NW
C = 32
NCH = SPW // C

_mesh = plsc.VectorSubcoreMesh(core_axis_name="c", subcore_axis_name="s")


@functools.partial(
    pl.kernel,
    mesh=_mesh,
    out_type=jax.ShapeDtypeStruct((B, D), jnp.float32),
    scratch_types=[
        pltpu.VMEM((F, NCH, C), jnp.int32),
        pltpu.VMEM((F * C, D), jnp.float32),
        pltpu.VMEM((C, D), jnp.float32),
        pltpu.SemaphoreType.DMA,
    ],
    compiler_params=pltpu.CompilerParams(use_tc_tiling_on_sc=False),
)
def _emb(x_hbm, tab_hbm, out_hbm, idx_v, rows_v, out_v, sem):
    wid = lax.axis_index("s") * NC + lax.axis_index("c")

    pltpu.sync_copy(x_hbm.at[:, wid], idx_v)

    @pl.loop(0, NCH)
    def _chunk(c):
        copies = [
            pltpu.async_copy(
                tab_hbm.at[f].at[idx_v.at[f, c]],
                rows_v.at[pl.ds(f * C, C), :],
                sem,
            )
            for f in range(F)
        ]
        for cp in copies:
            cp.wait()

        @pl.loop(0, C)
        def _acc(s):
            for v in range(D // L):
                sl = pl.ds(v * L, L)
                a = rows_v[s, sl]
                for f in range(1, F):
                    a = a + rows_v[f * C + s, sl]
                out_v[s, sl] = a

        pltpu.sync_copy(out_v, out_hbm.at[pl.ds(wid * SPW + c * C, C), :])


def kernel(x, tables):
    xt = x.astype(jnp.int32).T.reshape(F, NW, NCH, C)
    return _emb(xt, tables)
